# Initial kernel scaffold; baseline (speedup 1.0000x reference)
import jax
import jax.numpy as jnp
from jax.experimental import pallas as pl


def kernel(x, edge_index, type_rep, Wr, br, W0, b0, Wg, bg, We, be):
    raise NotImplementedError("write your pallas kernel here")


# speedup vs baseline: 1.1533x; 1.1533x over previous
import functools

import jax
import jax.numpy as jnp
from jax import lax
from jax.experimental import pallas as pl
from jax.experimental.pallas import tpu as pltpu
from jax.experimental.pallas import tpu_sc as plsc

NC = 2
NS = 16
NW = NC * NS
L = 16

G = 80


def _sc_mesh():
    return plsc.VectorSubcoreMesh(
        core_axis_name="c", subcore_axis_name="s", num_cores=NC,
        num_subcores=NS)


def _deg_body(ei_hbm, out_hbm, dst_v, deg_v):
    cid = lax.axis_index("c")
    sid = lax.axis_index("s")
    wid = sid * NC + cid
    e_total = ei_hbm.shape[0] // 2
    chunk = e_total // NW
    base = wid * chunk
    pltpu.sync_copy(ei_hbm.at[pl.ds(e_total + base, chunk)], dst_v)

    n_nodes = deg_v.shape[0]
    zeros = jnp.zeros((L,), jnp.float32)

    def zero_body(i, c):
        deg_v[pl.ds(i * L, L)] = zeros
        return c

    lax.fori_loop(0, n_nodes // L, zero_body, 0)

    iota = lax.iota(jnp.int32, L)

    def scat_body(i, c):
        idx = dst_v[pl.ds(i * L, L)]
        cnt = jnp.zeros((L,), jnp.float32)
        lastpos = jnp.full((L,), -1, jnp.int32)
        for j in range(L):
            eq = idx == idx[j]
            cnt = cnt + jnp.where(eq, 1.0, 0.0)
            lastpos = jnp.where(eq, j, lastpos)
        plsc.addupdate_scatter(deg_v, [idx], cnt, mask=lastpos == iota)
        return c

    lax.fori_loop(0, chunk // L, scat_body, 0)
    pltpu.sync_copy(deg_v, out_hbm.at[wid])


def _deg_call(ei_flat, n_nodes):
    e_total = ei_flat.shape[0] // 2
    chunk = e_total // NW
    return pl.kernel(
        _deg_body,
        out_type=jax.ShapeDtypeStruct((NW, n_nodes), jnp.float32),
        mesh=_sc_mesh(),
        compiler_params=pltpu.CompilerParams(needs_layout_passes=False),
        scratch_types=[
            pltpu.VMEM((chunk,), jnp.int32),
            pltpu.VMEM((n_nodes,), jnp.float32),
        ],
    )(ei_flat)


def _h_body(x_ref, wr_ref, br_ref, w0_ref, b0_ref, dp_ref, h_ref):
    wc = wr_ref[0] + wr_ref[1]
    brs = br_ref[0] + br_ref[1]
    xb = x_ref[...]
    rel = lax.dot_general(xb, wc, (((1,), (1,)), ((), ())),
                          preferred_element_type=jnp.float32) + brs
    z = lax.dot_general(xb, w0_ref[...], (((1,), (1,)), ((), ())),
                        preferred_element_type=jnp.float32) + b0_ref[...]
    ones = jnp.ones((NW, 1), jnp.float32)
    deg_col = lax.dot_general(dp_ref[...], ones, (((0,), (0,)), ((), ())),
                              preferred_element_type=jnp.float32)
    h_ref[...] = jnp.maximum(deg_col * rel + z, 0.0)


def _h_call(x, Wr, br, W0, b0, deg_part):
    n, d = x.shape
    return pl.pallas_call(
        _h_body,
        out_shape=jax.ShapeDtypeStruct((n, d), jnp.float32),
    )(x, Wr, br, W0, b0, deg_part)


def _edge_body(h_hbm, src_hbm, dst_hbm, wg_hbm, we_hbm, a_hbm, b_hbm,
               sidx, didx, hs0, hs1, hd0, hd1, we_tmp, wvec, av, bv,
               sem_s0, sem_d0, sem_s1, sem_d1):
    cid = lax.axis_index("c")
    sid = lax.axis_index("s")
    wid = sid * NC + cid
    ngroups = sidx.shape[0]
    chunk = ngroups * G
    base = wid * chunk
    d_model = h_hbm.shape[1]

    pltpu.sync_copy(src_hbm.at[wid], sidx)
    pltpu.sync_copy(dst_hbm.at[wid], didx)

    pltpu.sync_copy(wg_hbm.at[0, pl.ds(d_model, d_model)], wvec.at[0])
    pltpu.sync_copy(we_hbm, we_tmp)
    for j in range(d_model // L):
        wvec[1, pl.ds(j * L, L)] = (we_tmp[0, 0, pl.ds(j * L, L)]
                                    + we_tmp[1, 0, pl.ds(j * L, L)])

    hs = (hs0, hs1)
    hd = (hd0, hd1)
    sems = ((sem_s0, sem_d0), (sem_s1, sem_d1))

    def issue(g, slot):
        pltpu.async_copy(h_hbm.at[sidx.at[g]], hs[slot], sems[slot][0])
        pltpu.async_copy(h_hbm.at[didx.at[g]], hd[slot], sems[slot][1])

    def wait(slot):
        pltpu.make_async_copy(h_hbm.at[sidx.at[0]], hs[slot],
                              sems[slot][0]).wait()
        pltpu.make_async_copy(h_hbm.at[didx.at[0]], hd[slot],
                              sems[slot][1]).wait()

    iota = lax.iota(jnp.int32, L)
    zeros = jnp.zeros((L,), jnp.float32)
    UN = 8
    nacc = 2 * UN

    def bf16_round(v):
        bits = plsc.bitcast(v, jnp.int32)
        lsb = lax.shift_right_logical(bits, 16) & 1
        bits = bits + 0x7FFF + lsb
        bits = bits & jnp.int32(-65536)
        return plsc.bitcast(bits, jnp.float32)

    def compute(g, slot):
        hsr = hs[slot]
        hdr = hd[slot]
        for sub in range(G // L):
            rows = sub * L + iota

            def d_body(dc, accs):
                out = list(accs)
                wa = bf16_round(wvec[0, pl.ds(dc * L, L)])
                wb = bf16_round(wvec[1, pl.ds(dc * L, L)])
                for k in range(L):
                    dd = dc * L + k
                    cols = jnp.full((L,), dd, jnp.int32)
                    sv = plsc.load_gather(hsr, [rows, cols])
                    dv = plsc.load_gather(hdr, [rows, cols])
                    u = bf16_round(sv * dv)
                    ka = k % UN
                    out[ka] = out[ka] + u * wa[k]
                    out[UN + ka] = out[UN + ka] + u * wb[k]
                return tuple(out)

            accs = lax.fori_loop(0, d_model // L, d_body, (zeros,) * nacc)
            aa = accs[0]
            bb = accs[UN]
            for k in range(1, UN):
                aa = aa + accs[k]
                bb = bb + accs[UN + k]
            off = g * G + sub * L
            av[pl.ds(off, L)] = aa
            bv[pl.ds(off, L)] = bb

    issue(0, 0)
    issue(1, 1)

    def outer(gp, c):
        for s2 in range(2):
            g = gp * 2 + s2
            wait(s2)
            compute(g, s2)

            @pl.when(g + 2 < ngroups)
            def _():
                issue(g + 2, s2)
        return c

    lax.fori_loop(0, ngroups // 2, outer, 0)
    if ngroups % 2:
        wait(0)
        compute(ngroups - 1, 0)

    pltpu.sync_copy(av, a_hbm.at[pl.ds(base, chunk)])
    pltpu.sync_copy(bv, b_hbm.at[pl.ds(base, chunk)])


def _edge_call(h, src3, dst3, Wg, We):
    n, d = h.shape
    ngroups = src3.shape[1]
    e_total = NW * ngroups * G
    out_sds = jax.ShapeDtypeStruct((e_total,), jnp.float32)
    return pl.kernel(
        _edge_body,
        out_type=(out_sds, out_sds),
        mesh=_sc_mesh(),
        compiler_params=pltpu.CompilerParams(needs_layout_passes=False),
        scratch_types=[
            pltpu.VMEM((ngroups, G), jnp.int32),
            pltpu.VMEM((ngroups, G), jnp.int32),
            pltpu.VMEM((G, d), jnp.float32),
            pltpu.VMEM((G, d), jnp.float32),
            pltpu.VMEM((G, d), jnp.float32),
            pltpu.VMEM((G, d), jnp.float32),
            pltpu.VMEM(We.shape, jnp.float32),
            pltpu.VMEM((2, d), jnp.float32),
            pltpu.VMEM((ngroups * G,), jnp.float32),
            pltpu.VMEM((ngroups * G,), jnp.float32),
            pltpu.SemaphoreType.DMA,
            pltpu.SemaphoreType.DMA,
            pltpu.SemaphoreType.DMA,
            pltpu.SemaphoreType.DMA,
        ],
    )(h, src3, dst3, Wg, We)


def _final_body(tr_ref, a_ref, b_ref, wg_ref, bg_ref, be_ref, out_ref):
    d_model = tr_ref.shape[2]
    wgt = wg_ref[0, 0:d_model]
    t = lax.dot_general(tr_ref[...], wgt, (((2,), (0,)), ((), ())),
                        preferred_element_type=jnp.float32)
    z = t + a_ref[0] + bg_ref[0, 0]
    delta = 1.0 / (1.0 + jnp.exp(-z))
    bsum = be_ref[0, 0] + be_ref[1, 0]
    out_ref[...] = (delta * (b_ref[0] + bsum))[None]


def _final_call(type_rep, a, b, Wg, bg2, be):
    e_total, d = type_rep.shape
    b3 = 25
    rows = e_total // d
    grid = rows // b3
    tr3 = type_rep.reshape(rows, d, d)
    a3 = a.reshape(grid, b3, d)
    b3v = b.reshape(grid, b3, d)
    out = pl.pallas_call(
        _final_body,
        grid=(grid,),
        in_specs=[
            pl.BlockSpec((b3, d, d), lambda i: (i, 0, 0)),
            pl.BlockSpec((1, b3, d), lambda i: (i, 0, 0)),
            pl.BlockSpec((1, b3, d), lambda i: (i, 0, 0)),
            pl.BlockSpec(Wg.shape, lambda i: (0, 0)),
            pl.BlockSpec(memory_space=pltpu.SMEM),
            pl.BlockSpec(memory_space=pltpu.SMEM),
        ],
        out_specs=pl.BlockSpec((1, b3, d), lambda i: (i, 0, 0)),
        out_shape=jax.ShapeDtypeStruct((grid, b3, d), jnp.float32),
    )(tr3, a3, b3v, Wg, bg2, be)
    return out.reshape(e_total, 1)


def kernel(x, edge_index, type_rep, Wr, br, W0, b0, Wg, bg, We, be):
    n = x.shape[0]
    ei_flat = edge_index.reshape(-1)
    e_total = edge_index.shape[1]
    ngroups = e_total // (NW * G)
    deg_part = _deg_call(ei_flat, n)
    h = _h_call(x, Wr, br, W0, b0, deg_part)
    src3 = edge_index[0].reshape(NW, ngroups, G)
    dst3 = edge_index[1].reshape(NW, ngroups, G)
    a, b = _edge_call(h, src3, dst3, Wg, We)
    return _final_call(type_rep, a, b, Wg, bg.reshape(1, 1), be)


# speedup vs baseline: 2.7587x; 2.3919x over previous
import functools

import jax
import jax.numpy as jnp
from jax import lax
from jax.experimental import pallas as pl
from jax.experimental.pallas import tpu as pltpu
from jax.experimental.pallas import tpu_sc as plsc

NC = 2
NS = 16
NW = NC * NS
L = 16

G = 80


def _sc_mesh():
    return plsc.VectorSubcoreMesh(
        core_axis_name="c", subcore_axis_name="s", num_cores=NC,
        num_subcores=NS)


def _deg_body(ei_hbm, out_hbm, dst_v, deg_v):
    cid = lax.axis_index("c")
    sid = lax.axis_index("s")
    wid = sid * NC + cid
    e_total = ei_hbm.shape[0] // 2
    chunk = e_total // NW
    base = wid * chunk
    pltpu.sync_copy(ei_hbm.at[pl.ds(e_total + base, chunk)], dst_v)

    n_nodes = deg_v.shape[0]
    zeros = jnp.zeros((L,), jnp.float32)

    def zero_body(i, c):
        deg_v[pl.ds(i * L, L)] = zeros
        return c

    lax.fori_loop(0, n_nodes // L, zero_body, 0)

    iota = lax.iota(jnp.int32, L)

    def scat_body(i, c):
        idx = dst_v[pl.ds(i * L, L)]
        cnt = jnp.zeros((L,), jnp.float32)
        lastpos = jnp.full((L,), -1, jnp.int32)
        for j in range(L):
            eq = idx == idx[j]
            cnt = cnt + jnp.where(eq, 1.0, 0.0)
            lastpos = jnp.where(eq, j, lastpos)
        plsc.addupdate_scatter(deg_v, [idx], cnt, mask=lastpos == iota)
        return c

    lax.fori_loop(0, chunk // L, scat_body, 0)
    pltpu.sync_copy(deg_v, out_hbm.at[wid])


def _deg_call(ei_flat, n_nodes):
    e_total = ei_flat.shape[0] // 2
    chunk = e_total // NW
    return pl.kernel(
        _deg_body,
        out_type=jax.ShapeDtypeStruct((NW, n_nodes), jnp.float32),
        mesh=_sc_mesh(),
        compiler_params=pltpu.CompilerParams(needs_layout_passes=False),
        scratch_types=[
            pltpu.VMEM((chunk,), jnp.int32),
            pltpu.VMEM((n_nodes,), jnp.float32),
        ],
    )(ei_flat)


def _h_body(x_ref, wr_ref, br_ref, w0_ref, b0_ref, dp_ref, h_ref):
    wc = wr_ref[0] + wr_ref[1]
    brs = br_ref[0] + br_ref[1]
    xb = x_ref[...]
    rel = lax.dot_general(xb, wc, (((1,), (1,)), ((), ())),
                          preferred_element_type=jnp.float32) + brs
    z = lax.dot_general(xb, w0_ref[...], (((1,), (1,)), ((), ())),
                        preferred_element_type=jnp.float32) + b0_ref[...]
    ones = jnp.ones((NW, 1), jnp.float32)
    deg_col = lax.dot_general(dp_ref[...], ones, (((0,), (0,)), ((), ())),
                              preferred_element_type=jnp.float32)
    h_ref[...] = jnp.maximum(deg_col * rel + z, 0.0)


def _h_call(x, Wr, br, W0, b0, deg_part):
    n, d = x.shape
    return pl.pallas_call(
        _h_body,
        out_shape=jax.ShapeDtypeStruct((n, d), jnp.float32),
    )(x, Wr, br, W0, b0, deg_part)


def _edge_body(h_hbm, src_hbm, dst_hbm, wg_hbm, we_hbm, a_hbm, b_hbm,
               sidx, didx, hs0, hs1, hd0, hd1, we_tmp, wvec, av, bv,
               paT_a, paT_b, sem_s0, sem_d0, sem_s1, sem_d1):
    cid = lax.axis_index("c")
    sid = lax.axis_index("s")
    wid = sid * NC + cid
    ngroups = sidx.shape[0]
    chunk = ngroups * G
    base = wid * chunk
    d_model = h_hbm.shape[1]

    pltpu.sync_copy(src_hbm.at[wid], sidx)
    pltpu.sync_copy(dst_hbm.at[wid], didx)

    pltpu.sync_copy(wg_hbm.at[0, pl.ds(d_model, d_model)], wvec.at[0])
    pltpu.sync_copy(we_hbm, we_tmp)
    for j in range(d_model // L):
        wvec[1, pl.ds(j * L, L)] = (we_tmp[0, 0, pl.ds(j * L, L)]
                                    + we_tmp[1, 0, pl.ds(j * L, L)])

    hs = (hs0, hs1)
    hd = (hd0, hd1)
    sems = ((sem_s0, sem_d0), (sem_s1, sem_d1))

    def issue(g, slot):
        pltpu.async_copy(h_hbm.at[sidx.at[g]], hs[slot], sems[slot][0])
        pltpu.async_copy(h_hbm.at[didx.at[g]], hd[slot], sems[slot][1])

    def wait(slot):
        pltpu.make_async_copy(h_hbm.at[sidx.at[0]], hs[slot],
                              sems[slot][0]).wait()
        pltpu.make_async_copy(h_hbm.at[didx.at[0]], hd[slot],
                              sems[slot][1]).wait()

    iota = lax.iota(jnp.int32, L)
    zeros = jnp.zeros((L,), jnp.float32)
    nj = d_model // L

    def bf16_round(v):
        bits = plsc.bitcast(v, jnp.int32)
        lsb = lax.shift_right_logical(bits, 16) & 1
        bits = bits + 0x7FFF + lsb
        bits = bits & jnp.int32(-65536)
        return plsc.bitcast(bits, jnp.float32)

    wa = [bf16_round(wvec[0, pl.ds(j * L, L)]) for j in range(nj)]
    wb = [bf16_round(wvec[1, pl.ds(j * L, L)]) for j in range(nj)]

    def compute(g, slot):
        hsr = hs[slot]
        hdr = hd[slot]
        for sub in range(G // L):
            def e_body(ep, c):
                for t in range(2):
                    el = ep * 2 + t
                    e = sub * L + el
                    acc_a = zeros
                    acc_b = zeros
                    for j in range(nj):
                        sv = hsr[e, pl.ds(j * L, L)]
                        dv = hdr[e, pl.ds(j * L, L)]
                        u = bf16_round(sv * dv)
                        acc_a = acc_a + u * wa[j]
                        acc_b = acc_b + u * wb[j]
                    cols = jnp.full((L,), el, jnp.int32)
                    plsc.store_scatter(paT_a, [iota, cols], acc_a)
                    plsc.store_scatter(paT_b, [iota, cols], acc_b)
                return c

            lax.fori_loop(0, L // 2, e_body, 0)
            suma = paT_a[0, :] + paT_a[1, :]
            sumb = paT_b[0, :] + paT_b[1, :]
            for l in range(2, L):
                suma = suma + paT_a[l, :]
                sumb = sumb + paT_b[l, :]
            off = g * G + sub * L
            av[pl.ds(off, L)] = suma
            bv[pl.ds(off, L)] = sumb

    issue(0, 0)
    issue(1, 1)

    def outer(gp, c):
        for s2 in range(2):
            g = gp * 2 + s2
            wait(s2)
            compute(g, s2)

            @pl.when(g + 2 < ngroups)
            def _():
                issue(g + 2, s2)
        return c

    lax.fori_loop(0, ngroups // 2, outer, 0)
    if ngroups % 2:
        wait(0)
        compute(ngroups - 1, 0)

    pltpu.sync_copy(av, a_hbm.at[pl.ds(base, chunk)])
    pltpu.sync_copy(bv, b_hbm.at[pl.ds(base, chunk)])


def _edge_call(h, src3, dst3, Wg, We):
    n, d = h.shape
    ngroups = src3.shape[1]
    e_total = NW * ngroups * G
    out_sds = jax.ShapeDtypeStruct((e_total,), jnp.float32)
    return pl.kernel(
        _edge_body,
        out_type=(out_sds, out_sds),
        mesh=_sc_mesh(),
        compiler_params=pltpu.CompilerParams(needs_layout_passes=False),
        scratch_types=[
            pltpu.VMEM((ngroups, G), jnp.int32),
            pltpu.VMEM((ngroups, G), jnp.int32),
            pltpu.VMEM((G, d), jnp.float32),
            pltpu.VMEM((G, d), jnp.float32),
            pltpu.VMEM((G, d), jnp.float32),
            pltpu.VMEM((G, d), jnp.float32),
            pltpu.VMEM(We.shape, jnp.float32),
            pltpu.VMEM((2, d), jnp.float32),
            pltpu.VMEM((ngroups * G,), jnp.float32),
            pltpu.VMEM((ngroups * G,), jnp.float32),
            pltpu.VMEM((L, L), jnp.float32),
            pltpu.VMEM((L, L), jnp.float32),
            pltpu.SemaphoreType.DMA,
            pltpu.SemaphoreType.DMA,
            pltpu.SemaphoreType.DMA,
            pltpu.SemaphoreType.DMA,
        ],
    )(h, src3, dst3, Wg, We)


def _final_body(tr_ref, a_ref, b_ref, wg_ref, bg_ref, be_ref, out_ref):
    d_model = tr_ref.shape[2]
    wgt = wg_ref[0, 0:d_model]
    t = lax.dot_general(tr_ref[...], wgt, (((2,), (0,)), ((), ())),
                        preferred_element_type=jnp.float32)
    z = t + a_ref[0] + bg_ref[0, 0]
    delta = 1.0 / (1.0 + jnp.exp(-z))
    bsum = be_ref[0, 0] + be_ref[1, 0]
    out_ref[...] = (delta * (b_ref[0] + bsum))[None]


def _final_call(type_rep, a, b, Wg, bg2, be):
    e_total, d = type_rep.shape
    b3 = 25
    rows = e_total // d
    grid = rows // b3
    tr3 = type_rep.reshape(rows, d, d)
    a3 = a.reshape(grid, b3, d)
    b3v = b.reshape(grid, b3, d)
    out = pl.pallas_call(
        _final_body,
        grid=(grid,),
        in_specs=[
            pl.BlockSpec((b3, d, d), lambda i: (i, 0, 0)),
            pl.BlockSpec((1, b3, d), lambda i: (i, 0, 0)),
            pl.BlockSpec((1, b3, d), lambda i: (i, 0, 0)),
            pl.BlockSpec(Wg.shape, lambda i: (0, 0)),
            pl.BlockSpec(memory_space=pltpu.SMEM),
            pl.BlockSpec(memory_space=pltpu.SMEM),
        ],
        out_specs=pl.BlockSpec((1, b3, d), lambda i: (i, 0, 0)),
        out_shape=jax.ShapeDtypeStruct((grid, b3, d), jnp.float32),
    )(tr3, a3, b3v, Wg, bg2, be)
    return out.reshape(e_total, 1)


def kernel(x, edge_index, type_rep, Wr, br, W0, b0, Wg, bg, We, be):
    n = x.shape[0]
    ei_flat = edge_index.reshape(-1)
    e_total = edge_index.shape[1]
    ngroups = e_total // (NW * G)
    deg_part = _deg_call(ei_flat, n)
    h = _h_call(x, Wr, br, W0, b0, deg_part)
    src3 = edge_index[0].reshape(NW, ngroups, G)
    dst3 = edge_index[1].reshape(NW, ngroups, G)
    a, b = _edge_call(h, src3, dst3, Wg, We)
    return _final_call(type_rep, a, b, Wg, bg.reshape(1, 1), be)


# speedup vs baseline: 3.0631x; 1.1104x over previous
import functools

import jax
import jax.numpy as jnp
from jax import lax
from jax.experimental import pallas as pl
from jax.experimental.pallas import tpu as pltpu
from jax.experimental.pallas import tpu_sc as plsc

NC = 2
NS = 16
NW = NC * NS
L = 16

G = 80


def _sc_mesh():
    return plsc.VectorSubcoreMesh(
        core_axis_name="c", subcore_axis_name="s", num_cores=NC,
        num_subcores=NS)


def _deg_body(ei_hbm, out_hbm, dst_v, deg_v):
    cid = lax.axis_index("c")
    sid = lax.axis_index("s")
    wid = sid * NC + cid
    e_total = ei_hbm.shape[0] // 2
    chunk = e_total // NW
    base = wid * chunk
    pltpu.sync_copy(ei_hbm.at[pl.ds(e_total + base, chunk)], dst_v)

    n_nodes = deg_v.shape[0]
    zeros = jnp.zeros((L,), jnp.float32)

    def zero_body(i, c):
        deg_v[pl.ds(i * L, L)] = zeros
        return c

    lax.fori_loop(0, n_nodes // L, zero_body, 0)

    iota = lax.iota(jnp.int32, L)

    def scat_body(i, c):
        idx = dst_v[pl.ds(i * L, L)]
        cnt = jnp.zeros((L,), jnp.float32)
        lastpos = jnp.full((L,), -1, jnp.int32)
        for j in range(L):
            eq = idx == idx[j]
            cnt = cnt + jnp.where(eq, 1.0, 0.0)
            lastpos = jnp.where(eq, j, lastpos)
        plsc.addupdate_scatter(deg_v, [idx], cnt, mask=lastpos == iota)
        return c

    lax.fori_loop(0, chunk // L, scat_body, 0)
    pltpu.sync_copy(deg_v, out_hbm.at[wid])


def _deg_call(ei_flat, n_nodes):
    e_total = ei_flat.shape[0] // 2
    chunk = e_total // NW
    return pl.kernel(
        _deg_body,
        out_type=jax.ShapeDtypeStruct((NW, n_nodes), jnp.float32),
        mesh=_sc_mesh(),
        compiler_params=pltpu.CompilerParams(needs_layout_passes=False),
        scratch_types=[
            pltpu.VMEM((chunk,), jnp.int32),
            pltpu.VMEM((n_nodes,), jnp.float32),
        ],
    )(ei_flat)


def _h_body(x_ref, wr_ref, br_ref, w0_ref, b0_ref, dp_ref, h_ref):
    wc = wr_ref[0] + wr_ref[1]
    brs = br_ref[0] + br_ref[1]
    xb = x_ref[...]
    rel = lax.dot_general(xb, wc, (((1,), (1,)), ((), ())),
                          preferred_element_type=jnp.float32) + brs
    z = lax.dot_general(xb, w0_ref[...], (((1,), (1,)), ((), ())),
                        preferred_element_type=jnp.float32) + b0_ref[...]
    ones = jnp.ones((NW, 1), jnp.float32)
    deg_col = lax.dot_general(dp_ref[...], ones, (((0,), (0,)), ((), ())),
                              preferred_element_type=jnp.float32)
    h_ref[...] = jnp.maximum(deg_col * rel + z, 0.0)


def _h_call(x, Wr, br, W0, b0, deg_part):
    n, d = x.shape
    return pl.pallas_call(
        _h_body,
        out_shape=jax.ShapeDtypeStruct((n, d), jnp.float32),
    )(x, Wr, br, W0, b0, deg_part)


def _edge_body(h_hbm, src_hbm, dst_hbm, wg_hbm, we_hbm, a_hbm, b_hbm,
               sidx, didx, hs0, hs1, hd0, hd1, we_tmp, wvec, av, bv,
               paT_a, paT_b, sem_s0, sem_d0, sem_s1, sem_d1):
    cid = lax.axis_index("c")
    sid = lax.axis_index("s")
    wid = sid * NC + cid
    ngroups = sidx.shape[0]
    chunk = ngroups * G
    base = wid * chunk
    d_model = h_hbm.shape[1]

    pltpu.sync_copy(src_hbm.at[wid], sidx)
    pltpu.sync_copy(dst_hbm.at[wid], didx)

    pltpu.sync_copy(wg_hbm.at[0, pl.ds(d_model, d_model)], wvec.at[0])
    pltpu.sync_copy(we_hbm, we_tmp)
    for j in range(d_model // L):
        wvec[1, pl.ds(j * L, L)] = (we_tmp[0, 0, pl.ds(j * L, L)]
                                    + we_tmp[1, 0, pl.ds(j * L, L)])

    hs = (hs0, hs1)
    hd = (hd0, hd1)
    sems = ((sem_s0, sem_d0), (sem_s1, sem_d1))

    def issue(g, slot):
        pltpu.async_copy(h_hbm.at[sidx.at[g]], hs[slot], sems[slot][0])
        pltpu.async_copy(h_hbm.at[didx.at[g]], hd[slot], sems[slot][1])

    def wait(slot):
        pltpu.make_async_copy(h_hbm.at[sidx.at[0]], hs[slot],
                              sems[slot][0]).wait()
        pltpu.make_async_copy(h_hbm.at[didx.at[0]], hd[slot],
                              sems[slot][1]).wait()

    iota = lax.iota(jnp.int32, L)
    zeros = jnp.zeros((L,), jnp.float32)
    nj = d_model // L

    def bf16_round(v):
        bits = plsc.bitcast(v, jnp.int32)
        lsb = lax.shift_right_logical(bits, 16) & 1
        bits = bits + 0x7FFF + lsb
        bits = bits & jnp.int32(-65536)
        return plsc.bitcast(bits, jnp.float32)

    def bf16_round_fast(v):
        bits = plsc.bitcast(v, jnp.int32) + 0x8000
        bits = bits & jnp.int32(-65536)
        return plsc.bitcast(bits, jnp.float32)

    wa = [bf16_round(wvec[0, pl.ds(j * L, L)]) for j in range(nj)]
    wb = [bf16_round(wvec[1, pl.ds(j * L, L)]) for j in range(nj)]

    def compute(g, slot):
        hsr = hs[slot]
        hdr = hd[slot]
        for sub in range(G // L):
            def e_body(ep, c):
                for t in range(2):
                    el = ep * 2 + t
                    e = sub * L + el
                    acc_a = zeros
                    acc_b = zeros
                    for j in range(nj):
                        sv = hsr[e, pl.ds(j * L, L)]
                        dv = hdr[e, pl.ds(j * L, L)]
                        u = bf16_round_fast(sv * dv)
                        acc_a = acc_a + u * wa[j]
                        acc_b = acc_b + u * wb[j]
                    cols = jnp.full((L,), el, jnp.int32)
                    plsc.store_scatter(paT_a, [iota, cols], acc_a)
                    plsc.store_scatter(paT_b, [iota, cols], acc_b)
                return c

            lax.fori_loop(0, L // 2, e_body, 0)
            suma = paT_a[0, :] + paT_a[1, :]
            sumb = paT_b[0, :] + paT_b[1, :]
            for l in range(2, L):
                suma = suma + paT_a[l, :]
                sumb = sumb + paT_b[l, :]
            off = g * G + sub * L
            av[pl.ds(off, L)] = suma
            bv[pl.ds(off, L)] = sumb

    issue(0, 0)
    issue(1, 1)

    def outer(gp, c):
        for s2 in range(2):
            g = gp * 2 + s2
            wait(s2)
            compute(g, s2)

            @pl.when(g + 2 < ngroups)
            def _():
                issue(g + 2, s2)
        return c

    lax.fori_loop(0, ngroups // 2, outer, 0)
    if ngroups % 2:
        wait(0)
        compute(ngroups - 1, 0)

    pltpu.sync_copy(av, a_hbm.at[pl.ds(base, chunk)])
    pltpu.sync_copy(bv, b_hbm.at[pl.ds(base, chunk)])


def _edge_call(h, src3, dst3, Wg, We):
    n, d = h.shape
    ngroups = src3.shape[1]
    e_total = NW * ngroups * G
    out_sds = jax.ShapeDtypeStruct((e_total,), jnp.float32)
    return pl.kernel(
        _edge_body,
        out_type=(out_sds, out_sds),
        mesh=_sc_mesh(),
        compiler_params=pltpu.CompilerParams(needs_layout_passes=False),
        scratch_types=[
            pltpu.VMEM((ngroups, G), jnp.int32),
            pltpu.VMEM((ngroups, G), jnp.int32),
            pltpu.VMEM((G, d), jnp.float32),
            pltpu.VMEM((G, d), jnp.float32),
            pltpu.VMEM((G, d), jnp.float32),
            pltpu.VMEM((G, d), jnp.float32),
            pltpu.VMEM(We.shape, jnp.float32),
            pltpu.VMEM((2, d), jnp.float32),
            pltpu.VMEM((ngroups * G,), jnp.float32),
            pltpu.VMEM((ngroups * G,), jnp.float32),
            pltpu.VMEM((L, L), jnp.float32),
            pltpu.VMEM((L, L), jnp.float32),
            pltpu.SemaphoreType.DMA,
            pltpu.SemaphoreType.DMA,
            pltpu.SemaphoreType.DMA,
            pltpu.SemaphoreType.DMA,
        ],
    )(h, src3, dst3, Wg, We)


def _final_body(tr_ref, a_ref, b_ref, wg_ref, bg_ref, be_ref, out_ref):
    d_model = tr_ref.shape[2]
    wgt = wg_ref[0, 0:d_model]
    t = lax.dot_general(tr_ref[...], wgt, (((2,), (0,)), ((), ())),
                        preferred_element_type=jnp.float32)
    z = t + a_ref[0] + bg_ref[0, 0]
    delta = 1.0 / (1.0 + jnp.exp(-z))
    bsum = be_ref[0, 0] + be_ref[1, 0]
    out_ref[...] = (delta * (b_ref[0] + bsum))[None]


def _final_call(type_rep, a, b, Wg, bg2, be):
    e_total, d = type_rep.shape
    b3 = 25
    rows = e_total // d
    grid = rows // b3
    tr3 = type_rep.reshape(rows, d, d)
    a3 = a.reshape(grid, b3, d)
    b3v = b.reshape(grid, b3, d)
    out = pl.pallas_call(
        _final_body,
        grid=(grid,),
        in_specs=[
            pl.BlockSpec((b3, d, d), lambda i: (i, 0, 0)),
            pl.BlockSpec((1, b3, d), lambda i: (i, 0, 0)),
            pl.BlockSpec((1, b3, d), lambda i: (i, 0, 0)),
            pl.BlockSpec(Wg.shape, lambda i: (0, 0)),
            pl.BlockSpec(memory_space=pltpu.SMEM),
            pl.BlockSpec(memory_space=pltpu.SMEM),
        ],
        out_specs=pl.BlockSpec((1, b3, d), lambda i: (i, 0, 0)),
        out_shape=jax.ShapeDtypeStruct((grid, b3, d), jnp.float32),
    )(tr3, a3, b3v, Wg, bg2, be)
    return out.reshape(e_total, 1)


def kernel(x, edge_index, type_rep, Wr, br, W0, b0, Wg, bg, We, be):
    n = x.shape[0]
    ei_flat = edge_index.reshape(-1)
    e_total = edge_index.shape[1]
    ngroups = e_total // (NW * G)
    deg_part = _deg_call(ei_flat, n)
    h = _h_call(x, Wr, br, W0, b0, deg_part)
    src3 = edge_index[0].reshape(NW, ngroups, G)
    dst3 = edge_index[1].reshape(NW, ngroups, G)
    a, b = _edge_call(h, src3, dst3, Wg, We)
    return _final_call(type_rep, a, b, Wg, bg.reshape(1, 1), be)


# speedup vs baseline: 4.1180x; 1.3444x over previous
import functools

import jax
import jax.numpy as jnp
from jax import lax
from jax.experimental import pallas as pl
from jax.experimental.pallas import tpu as pltpu
from jax.experimental.pallas import tpu_sc as plsc

NC = 2
NS = 16
NW = NC * NS
L = 16

G = 80


def _sc_mesh():
    return plsc.VectorSubcoreMesh(
        core_axis_name="c", subcore_axis_name="s", num_cores=NC,
        num_subcores=NS)


def _deg_body(ei_hbm, out_hbm, dst_v, deg_v):
    cid = lax.axis_index("c")
    sid = lax.axis_index("s")
    wid = sid * NC + cid
    e_total = ei_hbm.shape[0] // 2
    chunk = e_total // NW
    base = wid * chunk
    pltpu.sync_copy(ei_hbm.at[pl.ds(e_total + base, chunk)], dst_v)

    n_nodes = deg_v.shape[0]
    zeros = jnp.zeros((L,), jnp.float32)

    def zero_body(i, c):
        deg_v[pl.ds(i * L, L)] = zeros
        return c

    lax.fori_loop(0, n_nodes // L, zero_body, 0)

    iota = lax.iota(jnp.int32, L)

    def scat_body(i, c):
        idx = dst_v[pl.ds(i * L, L)]
        cnt = jnp.zeros((L,), jnp.float32)
        lastpos = jnp.full((L,), -1, jnp.int32)
        for j in range(L):
            eq = idx == idx[j]
            cnt = cnt + jnp.where(eq, 1.0, 0.0)
            lastpos = jnp.where(eq, j, lastpos)
        plsc.addupdate_scatter(deg_v, [idx], cnt, mask=lastpos == iota)
        return c

    lax.fori_loop(0, chunk // L, scat_body, 0)
    pltpu.sync_copy(deg_v, out_hbm.at[wid])


def _deg_call(ei_flat, n_nodes):
    e_total = ei_flat.shape[0] // 2
    chunk = e_total // NW
    return pl.kernel(
        _deg_body,
        out_type=jax.ShapeDtypeStruct((NW, n_nodes), jnp.float32),
        mesh=_sc_mesh(),
        compiler_params=pltpu.CompilerParams(needs_layout_passes=False),
        scratch_types=[
            pltpu.VMEM((chunk,), jnp.int32),
            pltpu.VMEM((n_nodes,), jnp.float32),
        ],
    )(ei_flat)


def _h_body(x_ref, wr_ref, br_ref, w0_ref, b0_ref, dp_ref, h_ref):
    wc = wr_ref[0] + wr_ref[1]
    brs = br_ref[0] + br_ref[1]
    xb = x_ref[...]
    rel = lax.dot_general(xb, wc, (((1,), (1,)), ((), ())),
                          preferred_element_type=jnp.float32) + brs
    z = lax.dot_general(xb, w0_ref[...], (((1,), (1,)), ((), ())),
                        preferred_element_type=jnp.float32) + b0_ref[...]
    ones = jnp.ones((NW, 1), jnp.float32)
    deg_col = lax.dot_general(dp_ref[...], ones, (((0,), (0,)), ((), ())),
                              preferred_element_type=jnp.float32)
    h_ref[...] = jnp.maximum(deg_col * rel + z, 0.0)


def _h_call(x, Wr, br, W0, b0, deg_part):
    n, d = x.shape
    return pl.pallas_call(
        _h_body,
        out_shape=jax.ShapeDtypeStruct((n, d), jnp.float32),
    )(x, Wr, br, W0, b0, deg_part)


def _edge_body(h_hbm, src_hbm, dst_hbm, wg_hbm, we_hbm, a_hbm, b_hbm,
               sidx, didx, hs0, hs1, hd0, hd1, we_tmp, wvec, av, bv,
               paT_a, paT_b, sem_s0, sem_d0, sem_s1, sem_d1):
    cid = lax.axis_index("c")
    sid = lax.axis_index("s")
    wid = sid * NC + cid
    ngroups = sidx.shape[0]
    chunk = ngroups * G
    base = wid * chunk
    d_model = h_hbm.shape[1]

    pltpu.sync_copy(src_hbm.at[wid], sidx)
    pltpu.sync_copy(dst_hbm.at[wid], didx)

    pltpu.sync_copy(wg_hbm.at[0, pl.ds(d_model, d_model)], wvec.at[0])
    pltpu.sync_copy(we_hbm, we_tmp)
    for j in range(d_model // L):
        wvec[1, pl.ds(j * L, L)] = (we_tmp[0, 0, pl.ds(j * L, L)]
                                    + we_tmp[1, 0, pl.ds(j * L, L)])

    hs = (hs0, hs1)
    hd = (hd0, hd1)
    sems = ((sem_s0, sem_d0), (sem_s1, sem_d1))

    def issue(g, slot):
        pltpu.async_copy(h_hbm.at[sidx.at[g]], hs[slot], sems[slot][0])
        pltpu.async_copy(h_hbm.at[didx.at[g]], hd[slot], sems[slot][1])

    def wait(slot):
        pltpu.make_async_copy(h_hbm.at[sidx.at[0]], hs[slot],
                              sems[slot][0]).wait()
        pltpu.make_async_copy(h_hbm.at[didx.at[0]], hd[slot],
                              sems[slot][1]).wait()

    iota = lax.iota(jnp.int32, L)
    zeros = jnp.zeros((L,), jnp.float32)
    nj = d_model // L

    def bf16_round(v):
        bits = plsc.bitcast(v, jnp.int32)
        lsb = lax.shift_right_logical(bits, 16) & 1
        bits = bits + 0x7FFF + lsb
        bits = bits & jnp.int32(-65536)
        return plsc.bitcast(bits, jnp.float32)

    def bf16_round_fast(v):
        bits = plsc.bitcast(v, jnp.int32) + 0x8000
        bits = bits & jnp.int32(-65536)
        return plsc.bitcast(bits, jnp.float32)

    wa = [bf16_round(wvec[0, pl.ds(j * L, L)]) for j in range(nj)]
    wb = [bf16_round(wvec[1, pl.ds(j * L, L)]) for j in range(nj)]

    def compute(g, slot):
        hsr = hs[slot]
        hdr = hd[slot]
        for sub in range(G // L):
            def e_body(ep, c):
                for t in range(2):
                    el = ep * 2 + t
                    e = sub * L + el
                    acc_a = zeros
                    acc_b = zeros
                    for j in range(nj):
                        sv = hsr[e, pl.ds(j * L, L)]
                        dv = hdr[e, pl.ds(j * L, L)]
                        u = bf16_round_fast(sv * dv)
                        acc_a = acc_a + u * wa[j]
                        acc_b = acc_b + u * wb[j]
                    cols = jnp.full((L,), el, jnp.int32)
                    plsc.store_scatter(paT_a, [iota, cols], acc_a)
                    plsc.store_scatter(paT_b, [iota, cols], acc_b)
                return c

            lax.fori_loop(0, L // 2, e_body, 0)
            suma = paT_a[0, :] + paT_a[1, :]
            sumb = paT_b[0, :] + paT_b[1, :]
            for l in range(2, L):
                suma = suma + paT_a[l, :]
                sumb = sumb + paT_b[l, :]
            off = g * G + sub * L
            av[pl.ds(off, L)] = suma
            bv[pl.ds(off, L)] = sumb

    issue(0, 0)
    issue(1, 1)

    def outer(gp, c):
        for s2 in range(2):
            g = gp * 2 + s2
            wait(s2)
            compute(g, s2)

            @pl.when(g + 2 < ngroups)
            def _():
                issue(g + 2, s2)
        return c

    lax.fori_loop(0, ngroups // 2, outer, 0)
    if ngroups % 2:
        wait(0)
        compute(ngroups - 1, 0)

    pltpu.sync_copy(av, a_hbm.at[pl.ds(base, chunk)])
    pltpu.sync_copy(bv, b_hbm.at[pl.ds(base, chunk)])


def _edge_call(h, src3, dst3, Wg, We):
    n, d = h.shape
    ngroups = src3.shape[1]
    e_total = NW * ngroups * G
    out_sds = jax.ShapeDtypeStruct((e_total,), jnp.float32)
    return pl.kernel(
        _edge_body,
        out_type=(out_sds, out_sds),
        mesh=_sc_mesh(),
        compiler_params=pltpu.CompilerParams(needs_layout_passes=False),
        scratch_types=[
            pltpu.VMEM((ngroups, G), jnp.int32),
            pltpu.VMEM((ngroups, G), jnp.int32),
            pltpu.VMEM((G, d), jnp.float32),
            pltpu.VMEM((G, d), jnp.float32),
            pltpu.VMEM((G, d), jnp.float32),
            pltpu.VMEM((G, d), jnp.float32),
            pltpu.VMEM(We.shape, jnp.float32),
            pltpu.VMEM((2, d), jnp.float32),
            pltpu.VMEM((ngroups * G,), jnp.float32),
            pltpu.VMEM((ngroups * G,), jnp.float32),
            pltpu.VMEM((L, L), jnp.float32),
            pltpu.VMEM((L, L), jnp.float32),
            pltpu.SemaphoreType.DMA,
            pltpu.SemaphoreType.DMA,
            pltpu.SemaphoreType.DMA,
            pltpu.SemaphoreType.DMA,
        ],
    )(h, src3, dst3, Wg, We)


def _gate_body(tr_ref, wg_ref, t_ref):
    d_model = tr_ref.shape[2]
    wgt = wg_ref[0, 0:d_model]
    t = lax.dot_general(tr_ref[...], wgt, (((2,), (0,)), ((), ())),
                        preferred_element_type=jnp.float32)
    t_ref[...] = t[None]


def _gate_call(type_rep, Wg):
    e_total, d = type_rep.shape
    b3 = 25
    rows = e_total // d
    grid = rows // b3
    tr3 = type_rep.reshape(rows, d, d)
    return pl.pallas_call(
        _gate_body,
        grid=(grid,),
        in_specs=[
            pl.BlockSpec((b3, d, d), lambda i: (i, 0, 0)),
            pl.BlockSpec(Wg.shape, lambda i: (0, 0)),
        ],
        out_specs=pl.BlockSpec((1, b3, d), lambda i: (i, 0, 0)),
        out_shape=jax.ShapeDtypeStruct((grid, b3, d), jnp.float32),
    )(tr3, Wg)


def _final_body(t_ref, a_ref, b_ref, bg_ref, be_ref, out_ref):
    z = t_ref[...] + a_ref[...] + bg_ref[0, 0]
    delta = 1.0 / (1.0 + jnp.exp(-z))
    bsum = be_ref[0, 0] + be_ref[1, 0]
    out_ref[...] = delta * (b_ref[...] + bsum)


def _final_call(t3, a, b, bg2, be):
    grid, b3, d = t3.shape
    e_total = grid * b3 * d
    a3 = a.reshape(grid, b3, d)
    b3v = b.reshape(grid, b3, d)
    gs = grid // 5
    out = pl.pallas_call(
        _final_body,
        grid=(gs,),
        in_specs=[
            pl.BlockSpec((5, b3, d), lambda i: (i, 0, 0)),
            pl.BlockSpec((5, b3, d), lambda i: (i, 0, 0)),
            pl.BlockSpec((5, b3, d), lambda i: (i, 0, 0)),
            pl.BlockSpec(memory_space=pltpu.SMEM),
            pl.BlockSpec(memory_space=pltpu.SMEM),
        ],
        out_specs=pl.BlockSpec((5, b3, d), lambda i: (i, 0, 0)),
        out_shape=jax.ShapeDtypeStruct((grid, b3, d), jnp.float32),
    )(t3, a3, b3v, bg2, be)
    return out.reshape(e_total, 1)


def kernel(x, edge_index, type_rep, Wr, br, W0, b0, Wg, bg, We, be):
    n = x.shape[0]
    ei_flat = edge_index.reshape(-1)
    e_total = edge_index.shape[1]
    ngroups = e_total // (NW * G)
    deg_part = _deg_call(ei_flat, n)
    t3 = _gate_call(type_rep, Wg)
    h = _h_call(x, Wr, br, W0, b0, deg_part)
    src3 = edge_index[0].reshape(NW, ngroups, G)
    dst3 = edge_index[1].reshape(NW, ngroups, G)
    a, b = _edge_call(h, src3, dst3, Wg, We)
    return _final_call(t3, a, b, bg.reshape(1, 1), be)


# speedup vs baseline: 4.2298x; 1.0272x over previous
import functools

import jax
import jax.numpy as jnp
from jax import lax
from jax.experimental import pallas as pl
from jax.experimental.pallas import tpu as pltpu
from jax.experimental.pallas import tpu_sc as plsc

NC = 2
NS = 16
NW = NC * NS
L = 16

G = 80


def _sc_mesh():
    return plsc.VectorSubcoreMesh(
        core_axis_name="c", subcore_axis_name="s", num_cores=NC,
        num_subcores=NS)


def _deg_body(ei_hbm, out_hbm, dst_v, deg_v):
    cid = lax.axis_index("c")
    sid = lax.axis_index("s")
    wid = sid * NC + cid
    e_total = ei_hbm.shape[0] // 2
    chunk = e_total // NW
    base = wid * chunk
    pltpu.sync_copy(ei_hbm.at[pl.ds(e_total + base, chunk)], dst_v)

    n_nodes = deg_v.shape[0]
    zeros = jnp.zeros((L,), jnp.float32)

    def zero_body(i, c):
        deg_v[pl.ds(i * L, L)] = zeros
        return c

    lax.fori_loop(0, n_nodes // L, zero_body, 0)

    iota = lax.iota(jnp.int32, L)

    def scat_body(i, c):
        idx = dst_v[pl.ds(i * L, L)]
        cnt = jnp.zeros((L,), jnp.float32)
        lastpos = jnp.full((L,), -1, jnp.int32)
        for j in range(L):
            eq = idx == idx[j]
            cnt = cnt + jnp.where(eq, 1.0, 0.0)
            lastpos = jnp.where(eq, j, lastpos)
        plsc.addupdate_scatter(deg_v, [idx], cnt, mask=lastpos == iota)
        return c

    lax.fori_loop(0, chunk // L, scat_body, 0)
    pltpu.sync_copy(deg_v, out_hbm.at[wid])


def _deg_call(ei_flat, n_nodes):
    e_total = ei_flat.shape[0] // 2
    chunk = e_total // NW
    return pl.kernel(
        _deg_body,
        out_type=jax.ShapeDtypeStruct((NW, n_nodes), jnp.float32),
        mesh=_sc_mesh(),
        compiler_params=pltpu.CompilerParams(needs_layout_passes=False),
        scratch_types=[
            pltpu.VMEM((chunk,), jnp.int32),
            pltpu.VMEM((n_nodes,), jnp.float32),
        ],
    )(ei_flat)


def _h_body(x_ref, wr_ref, br_ref, w0_ref, b0_ref, dp_ref, h_ref):
    wc = wr_ref[0] + wr_ref[1]
    brs = br_ref[0] + br_ref[1]
    xb = x_ref[...]
    rel = lax.dot_general(xb, wc, (((1,), (1,)), ((), ())),
                          preferred_element_type=jnp.float32) + brs
    z = lax.dot_general(xb, w0_ref[...], (((1,), (1,)), ((), ())),
                        preferred_element_type=jnp.float32) + b0_ref[...]
    ones = jnp.ones((NW, 1), jnp.float32)
    deg_col = lax.dot_general(dp_ref[...], ones, (((0,), (0,)), ((), ())),
                              preferred_element_type=jnp.float32)
    h_ref[...] = jnp.maximum(deg_col * rel + z, 0.0)


def _h_call(x, Wr, br, W0, b0, deg_part):
    n, d = x.shape
    return pl.pallas_call(
        _h_body,
        out_shape=jax.ShapeDtypeStruct((n, d), jnp.float32),
    )(x, Wr, br, W0, b0, deg_part)


def _edge_body(h_hbm, src_hbm, dst_hbm, wg_hbm, we_hbm, a_hbm, b_hbm,
               sidx, didx, hs0, hs1, hd0, hd1, we_tmp, wvec, av, bv,
               paT_a, paT_b, sem_s0, sem_d0, sem_s1, sem_d1):
    cid = lax.axis_index("c")
    sid = lax.axis_index("s")
    wid = sid * NC + cid
    ngroups = sidx.shape[0]
    chunk = ngroups * G
    base = wid * chunk
    d_model = h_hbm.shape[1]

    pltpu.sync_copy(src_hbm.at[wid], sidx)
    pltpu.sync_copy(dst_hbm.at[wid], didx)

    pltpu.sync_copy(wg_hbm.at[0, pl.ds(d_model, d_model)], wvec.at[0])
    pltpu.sync_copy(we_hbm, we_tmp)
    for j in range(d_model // L):
        wvec[1, pl.ds(j * L, L)] = (we_tmp[0, 0, pl.ds(j * L, L)]
                                    + we_tmp[1, 0, pl.ds(j * L, L)])

    hs = (hs0, hs1)
    hd = (hd0, hd1)
    sems = ((sem_s0, sem_d0), (sem_s1, sem_d1))

    def issue(g, slot):
        pltpu.async_copy(h_hbm.at[sidx.at[g]], hs[slot], sems[slot][0])
        pltpu.async_copy(h_hbm.at[didx.at[g]], hd[slot], sems[slot][1])

    def wait(slot):
        pltpu.make_async_copy(h_hbm.at[sidx.at[0]], hs[slot],
                              sems[slot][0]).wait()
        pltpu.make_async_copy(h_hbm.at[didx.at[0]], hd[slot],
                              sems[slot][1]).wait()

    iota = lax.iota(jnp.int32, L)
    zeros = jnp.zeros((L,), jnp.float32)
    nj = d_model // L

    def bf16_round(v):
        bits = plsc.bitcast(v, jnp.int32)
        lsb = lax.shift_right_logical(bits, 16) & 1
        bits = bits + 0x7FFF + lsb
        bits = bits & jnp.int32(-65536)
        return plsc.bitcast(bits, jnp.float32)

    def bf16_round_fast(v):
        bits = plsc.bitcast(v, jnp.int32) + 0x8000
        bits = bits & jnp.int32(-65536)
        return plsc.bitcast(bits, jnp.float32)

    wa = [bf16_round(wvec[0, pl.ds(j * L, L)]) for j in range(nj)]
    wb = [bf16_round(wvec[1, pl.ds(j * L, L)]) for j in range(nj)]

    def compute(g, slot):
        hsr = hs[slot]
        hdr = hd[slot]
        for sub in range(G // L):
            def e_body(ep, c):
                for t in range(2):
                    el = ep * 2 + t
                    e = sub * L + el
                    pa = []
                    pb = []
                    for j in range(nj):
                        sv = hsr[e, pl.ds(j * L, L)]
                        dv = hdr[e, pl.ds(j * L, L)]
                        u = bf16_round_fast(sv * dv)
                        pa.append(u * wa[j])
                        pb.append(u * wb[j])
                    while len(pa) > 1:
                        pa = [pa[i] + pa[i + 1] for i in range(0, len(pa), 2)]
                        pb = [pb[i] + pb[i + 1] for i in range(0, len(pb), 2)]
                    acc_a = pa[0]
                    acc_b = pb[0]
                    cols = jnp.full((L,), el, jnp.int32)
                    plsc.store_scatter(paT_a, [iota, cols], acc_a)
                    plsc.store_scatter(paT_b, [iota, cols], acc_b)
                return c

            lax.fori_loop(0, L // 2, e_body, 0)
            suma = paT_a[0, :] + paT_a[1, :]
            sumb = paT_b[0, :] + paT_b[1, :]
            for l in range(2, L):
                suma = suma + paT_a[l, :]
                sumb = sumb + paT_b[l, :]
            off = g * G + sub * L
            av[pl.ds(off, L)] = suma
            bv[pl.ds(off, L)] = sumb

    issue(0, 0)
    issue(1, 1)

    def outer(gp, c):
        for s2 in range(2):
            g = gp * 2 + s2
            wait(s2)
            compute(g, s2)

            @pl.when(g + 2 < ngroups)
            def _():
                issue(g + 2, s2)
        return c

    lax.fori_loop(0, ngroups // 2, outer, 0)
    if ngroups % 2:
        wait(0)
        compute(ngroups - 1, 0)

    pltpu.sync_copy(av, a_hbm.at[pl.ds(base, chunk)])
    pltpu.sync_copy(bv, b_hbm.at[pl.ds(base, chunk)])


def _edge_call(h, src3, dst3, Wg, We):
    n, d = h.shape
    ngroups = src3.shape[1]
    e_total = NW * ngroups * G
    out_sds = jax.ShapeDtypeStruct((e_total,), jnp.float32)
    return pl.kernel(
        _edge_body,
        out_type=(out_sds, out_sds),
        mesh=_sc_mesh(),
        compiler_params=pltpu.CompilerParams(needs_layout_passes=False),
        scratch_types=[
            pltpu.VMEM((ngroups, G), jnp.int32),
            pltpu.VMEM((ngroups, G), jnp.int32),
            pltpu.VMEM((G, d), jnp.float32),
            pltpu.VMEM((G, d), jnp.float32),
            pltpu.VMEM((G, d), jnp.float32),
            pltpu.VMEM((G, d), jnp.float32),
            pltpu.VMEM(We.shape, jnp.float32),
            pltpu.VMEM((2, d), jnp.float32),
            pltpu.VMEM((ngroups * G,), jnp.float32),
            pltpu.VMEM((ngroups * G,), jnp.float32),
            pltpu.VMEM((L, L), jnp.float32),
            pltpu.VMEM((L, L), jnp.float32),
            pltpu.SemaphoreType.DMA,
            pltpu.SemaphoreType.DMA,
            pltpu.SemaphoreType.DMA,
            pltpu.SemaphoreType.DMA,
        ],
    )(h, src3, dst3, Wg, We)


def _gate_body(tr_ref, wg_ref, t_ref):
    d_model = tr_ref.shape[2]
    wgt = wg_ref[0, 0:d_model]
    t = lax.dot_general(tr_ref[...], wgt, (((2,), (0,)), ((), ())),
                        preferred_element_type=jnp.float32)
    t_ref[...] = t[None]


def _gate_call(type_rep, Wg):
    e_total, d = type_rep.shape
    b3 = 25
    rows = e_total // d
    grid = rows // b3
    tr3 = type_rep.reshape(rows, d, d)
    return pl.pallas_call(
        _gate_body,
        grid=(grid,),
        in_specs=[
            pl.BlockSpec((b3, d, d), lambda i: (i, 0, 0)),
            pl.BlockSpec(Wg.shape, lambda i: (0, 0)),
        ],
        out_specs=pl.BlockSpec((1, b3, d), lambda i: (i, 0, 0)),
        out_shape=jax.ShapeDtypeStruct((grid, b3, d), jnp.float32),
    )(tr3, Wg)


def _final_body(t_ref, a_ref, b_ref, bg_ref, be_ref, out_ref):
    z = t_ref[...] + a_ref[...] + bg_ref[0, 0]
    delta = 1.0 / (1.0 + jnp.exp(-z))
    bsum = be_ref[0, 0] + be_ref[1, 0]
    out_ref[...] = delta * (b_ref[...] + bsum)


def _final_call(t3, a, b, bg2, be):
    grid, b3, d = t3.shape
    e_total = grid * b3 * d
    a3 = a.reshape(grid, b3, d)
    b3v = b.reshape(grid, b3, d)
    gs = grid // 5
    out = pl.pallas_call(
        _final_body,
        grid=(gs,),
        in_specs=[
            pl.BlockSpec((5, b3, d), lambda i: (i, 0, 0)),
            pl.BlockSpec((5, b3, d), lambda i: (i, 0, 0)),
            pl.BlockSpec((5, b3, d), lambda i: (i, 0, 0)),
            pl.BlockSpec(memory_space=pltpu.SMEM),
            pl.BlockSpec(memory_space=pltpu.SMEM),
        ],
        out_specs=pl.BlockSpec((5, b3, d), lambda i: (i, 0, 0)),
        out_shape=jax.ShapeDtypeStruct((grid, b3, d), jnp.float32),
    )(t3, a3, b3v, bg2, be)
    return out.reshape(e_total, 1)


def kernel(x, edge_index, type_rep, Wr, br, W0, b0, Wg, bg, We, be):
    n = x.shape[0]
    ei_flat = edge_index.reshape(-1)
    e_total = edge_index.shape[1]
    ngroups = e_total // (NW * G)
    deg_part = _deg_call(ei_flat, n)
    t3 = _gate_call(type_rep, Wg)
    h = _h_call(x, Wr, br, W0, b0, deg_part)
    src3 = edge_index[0].reshape(NW, ngroups, G)
    dst3 = edge_index[1].reshape(NW, ngroups, G)
    a, b = _edge_call(h, src3, dst3, Wg, We)
    return _final_call(t3, a, b, bg.reshape(1, 1), be)
